# Initial kernel scaffold; baseline (speedup 1.0000x reference)
import jax
import jax.numpy as jnp
from jax.experimental import pallas as pl


def kernel(input_ids, token_type_ids, W_E, W_pos, W_token_type, ln_w, ln_b):
    raise NotImplementedError("write your pallas kernel here")


# speedup vs baseline: 1.3937x; 1.3937x over previous
import functools

import jax
import jax.numpy as jnp
from jax import lax
from jax.experimental import pallas as pl
from jax.experimental.pallas import tpu as pltpu
from jax.experimental.pallas import tpu_sc as plsc

_EPS = 1e-5

_NC = 2
_NS = 16
_NW = _NC * _NS
_CHUNK = 64


def _sc_gather(table, flat_idx):
    n = flat_idx.shape[0]
    d = table.shape[1]
    b_per_w = n // _NW
    n_chunks = b_per_w // _CHUNK
    mesh = plsc.VectorSubcoreMesh(core_axis_name="c", subcore_axis_name="s")

    @functools.partial(
        pl.kernel,
        out_type=jax.ShapeDtypeStruct((n, d), table.dtype),
        mesh=mesh,
        scratch_types=[
            pltpu.VMEM((b_per_w,), jnp.int32),
            pltpu.VMEM((_CHUNK, d), jnp.float32),
            pltpu.SemaphoreType.DMA,
        ],
    )
    def gather_kernel(table_hbm, idx_hbm, out_hbm, idx_v, rows_v, sem):
        wid = lax.axis_index("s") * _NC + lax.axis_index("c")
        base = wid * b_per_w
        pltpu.sync_copy(idx_hbm.at[pl.ds(base, b_per_w)], idx_v)
        for c in range(n_chunks):
            pltpu.async_copy(
                table_hbm.at[idx_v.at[pl.ds(c * _CHUNK, _CHUNK)]], rows_v, sem
            ).wait()
            pltpu.sync_copy(rows_v, out_hbm.at[pl.ds(base + c * _CHUNK, _CHUNK)])

    return gather_kernel(table, flat_idx)


def _ln_body(gath_ref, pos_ref, tt_ref, wtt_ref, lnw_ref, lnb_ref, out_ref):
    x = gath_ref[...]
    tt = tt_ref[...]
    w0 = wtt_ref[0, :][None, :]
    w1 = wtt_ref[1, :][None, :]
    tte = w0 + tt * (w1 - w0)
    x = x + pos_ref[...] + tte
    mu = jnp.mean(x, axis=-1, keepdims=True)
    xc = x - mu
    var = jnp.mean(xc * xc, axis=-1, keepdims=True)
    y = xc * lax.rsqrt(var + _EPS)
    out_ref[...] = y * lnw_ref[...] + lnb_ref[...]


def _tc_add_ln(gathered, token_type_ids, W_pos, W_token_type, ln_w, ln_b):
    batch, seq = token_type_ids.shape
    d = gathered.shape[-1]
    rows_per_blk = 256
    seq_blks = seq // rows_per_blk
    n_blks = batch * seq_blks

    ttf = token_type_ids.reshape(batch * seq, 1).astype(jnp.float32)
    lnw2 = ln_w.reshape(1, d)
    lnb2 = ln_b.reshape(1, d)

    out = pl.pallas_call(
        _ln_body,
        grid=(seq_blks, batch),
        in_specs=[
            pl.BlockSpec((rows_per_blk, d), lambda j, b: (b * seq_blks + j, 0)),
            pl.BlockSpec((rows_per_blk, d), lambda j, b: (j, 0)),
            pl.BlockSpec((rows_per_blk, 1), lambda j, b: (b * seq_blks + j, 0)),
            pl.BlockSpec((2, d), lambda j, b: (0, 0)),
            pl.BlockSpec((1, d), lambda j, b: (0, 0)),
            pl.BlockSpec((1, d), lambda j, b: (0, 0)),
        ],
        out_specs=pl.BlockSpec((rows_per_blk, d), lambda j, b: (b * seq_blks + j, 0)),
        out_shape=jax.ShapeDtypeStruct((batch * seq, d), gathered.dtype),
    )(gathered, W_pos, ttf, W_token_type, lnw2, lnb2)
    return out.reshape(batch, seq, d)


@jax.jit
def kernel(input_ids, token_type_ids, W_E, W_pos, W_token_type, ln_w, ln_b):
    batch, seq = input_ids.shape
    flat_idx = input_ids.reshape(batch * seq).astype(jnp.int32)
    gathered = _sc_gather(W_E, flat_idx)
    return _tc_add_ln(gathered, token_type_ids, W_pos, W_token_type, ln_w, ln_b)


# speedup vs baseline: 1.3959x; 1.0016x over previous
import functools

import jax
import jax.numpy as jnp
from jax import lax
from jax.experimental import pallas as pl
from jax.experimental.pallas import tpu as pltpu
from jax.experimental.pallas import tpu_sc as plsc

_EPS = 1e-5

_NC = 2
_NS = 16
_NW = _NC * _NS
_CHUNK = 64


def _sc_gather(table, flat_idx):
    n = flat_idx.shape[0]
    d = table.shape[1]
    b_per_w = n // _NW
    n_chunks = b_per_w // _CHUNK
    mesh = plsc.VectorSubcoreMesh(core_axis_name="c", subcore_axis_name="s")

    @functools.partial(
        pl.kernel,
        out_type=jax.ShapeDtypeStruct((n, d), table.dtype),
        mesh=mesh,
        scratch_types=[
            pltpu.VMEM((b_per_w,), jnp.int32),
            pltpu.VMEM((_CHUNK, d), jnp.float32),
            pltpu.SemaphoreType.DMA,
        ],
    )
    def gather_kernel(table_hbm, idx_hbm, out_hbm, idx_v, rows_v, sem):
        wid = lax.axis_index("s") * _NC + lax.axis_index("c")
        base = wid * b_per_w
        pltpu.sync_copy(idx_hbm.at[pl.ds(base, b_per_w)], idx_v)
        for c in range(n_chunks):
            pltpu.async_copy(
                table_hbm.at[idx_v.at[pl.ds(c * _CHUNK, _CHUNK)]], rows_v, sem
            ).wait()
            pltpu.sync_copy(rows_v, out_hbm.at[pl.ds(base + c * _CHUNK, _CHUNK)])

    return gather_kernel(table, flat_idx)


def _ln_body(gath_ref, pos_ref, tt_ref, wtt_ref, lnw_ref, lnb_ref, out_ref):
    x = gath_ref[...]
    tt = tt_ref[...]
    w0 = wtt_ref[0, :][None, :]
    w1 = wtt_ref[1, :][None, :]
    tte = w0 + tt * (w1 - w0)
    x = x + pos_ref[...] + tte
    mu = jnp.mean(x, axis=-1, keepdims=True)
    xc = x - mu
    var = jnp.mean(xc * xc, axis=-1, keepdims=True)
    y = xc * lax.rsqrt(var + _EPS)
    out_ref[...] = y * lnw_ref[...] + lnb_ref[...]


def _tc_add_ln(gathered, token_type_ids, W_pos, W_token_type, ln_w, ln_b):
    batch, seq = token_type_ids.shape
    d = gathered.shape[-1]
    rows_per_blk = 256
    seq_blks = seq // rows_per_blk
    n_blks = batch * seq_blks

    ttf = token_type_ids.reshape(batch * seq, 1).astype(jnp.float32)
    lnw2 = ln_w.reshape(1, d)
    lnb2 = ln_b.reshape(1, d)

    out = pl.pallas_call(
        _ln_body,
        grid=(seq_blks, batch),
        in_specs=[
            pl.BlockSpec((rows_per_blk, d), lambda j, b: (b * seq_blks + j, 0)),
            pl.BlockSpec((rows_per_blk, d), lambda j, b: (j, 0)),
            pl.BlockSpec((rows_per_blk, 1), lambda j, b: (b * seq_blks + j, 0)),
            pl.BlockSpec((2, d), lambda j, b: (0, 0)),
            pl.BlockSpec((1, d), lambda j, b: (0, 0)),
            pl.BlockSpec((1, d), lambda j, b: (0, 0)),
        ],
        out_specs=pl.BlockSpec((rows_per_blk, d), lambda j, b: (b * seq_blks + j, 0)),
        out_shape=jax.ShapeDtypeStruct((batch * seq, d), gathered.dtype),
        compiler_params=pltpu.CompilerParams(
            dimension_semantics=("parallel", "parallel")
        ),
    )(gathered, W_pos, ttf, W_token_type, lnw2, lnb2)
    return out.reshape(batch, seq, d)


@jax.jit
def kernel(input_ids, token_type_ids, W_E, W_pos, W_token_type, ln_w, ln_b):
    batch, seq = input_ids.shape
    flat_idx = input_ids.reshape(batch * seq).astype(jnp.int32)
    gathered = _sc_gather(W_E, flat_idx)
    return _tc_add_ln(gathered, token_type_ids, W_pos, W_token_type, ln_w, ln_b)


# speedup vs baseline: 1.6032x; 1.1485x over previous
import functools

import jax
import jax.numpy as jnp
from jax import lax
from jax.experimental import pallas as pl
from jax.experimental.pallas import tpu as pltpu
from jax.experimental.pallas import tpu_sc as plsc

_EPS = 1e-5

_NC = 2
_NS = 16
_NW = _NC * _NS
_CHUNK = 64


def _sc_gather(table, flat_idx):
    n = flat_idx.shape[0]
    d = table.shape[1]
    b_per_w = n // _NW
    n_chunks = b_per_w // _CHUNK
    mesh = plsc.VectorSubcoreMesh(core_axis_name="c", subcore_axis_name="s")

    @functools.partial(
        pl.kernel,
        out_type=jax.ShapeDtypeStruct((n, d), table.dtype),
        mesh=mesh,
        scratch_types=[
            pltpu.VMEM((b_per_w,), jnp.int32),
            pltpu.VMEM((_CHUNK, d), jnp.float32),
            pltpu.SemaphoreType.DMA,
        ],
    )
    def gather_kernel(table_hbm, idx_hbm, out_hbm, idx_v, rows_v, sem):
        wid = lax.axis_index("s") * _NC + lax.axis_index("c")
        base = wid * b_per_w
        pltpu.sync_copy(idx_hbm.at[pl.ds(base, b_per_w)], idx_v)
        for c in range(n_chunks):
            pltpu.async_copy(
                table_hbm.at[idx_v.at[pl.ds(c * _CHUNK, _CHUNK)]], rows_v, sem
            ).wait()
            pltpu.sync_copy(rows_v, out_hbm.at[pl.ds(base + c * _CHUNK, _CHUNK)])

    return gather_kernel(table, flat_idx)


def _ln_body(gath_ref, pos_ref, tt_ref, wtt_ref, lnw_ref, lnb_ref, out_ref):
    x = gath_ref[...]
    tt = tt_ref[...].astype(jnp.float32)
    w0 = wtt_ref[0, :][None, :]
    w1 = wtt_ref[1, :][None, :]
    tte = w0 + tt * (w1 - w0)
    x = x + pos_ref[...] + tte
    mu = jnp.mean(x, axis=-1, keepdims=True)
    xc = x - mu
    var = jnp.mean(xc * xc, axis=-1, keepdims=True)
    y = xc * lax.rsqrt(var + _EPS)
    out_ref[...] = y * lnw_ref[...] + lnb_ref[...]


def _tc_add_ln(gathered, token_type_ids, W_pos, W_token_type, ln_w, ln_b):
    batch, seq = token_type_ids.shape
    d = gathered.shape[-1]
    rows_per_blk = 512
    seq_blks = seq // rows_per_blk

    tt2 = token_type_ids.reshape(batch * seq, 1)
    lnw2 = ln_w.reshape(1, d)
    lnb2 = ln_b.reshape(1, d)

    out = pl.pallas_call(
        _ln_body,
        grid=(seq_blks, batch),
        in_specs=[
            pl.BlockSpec((rows_per_blk, d), lambda j, b: (b * seq_blks + j, 0)),
            pl.BlockSpec((rows_per_blk, d), lambda j, b: (j, 0)),
            pl.BlockSpec((rows_per_blk, 1), lambda j, b: (b * seq_blks + j, 0)),
            pl.BlockSpec((2, d), lambda j, b: (0, 0)),
            pl.BlockSpec((1, d), lambda j, b: (0, 0)),
            pl.BlockSpec((1, d), lambda j, b: (0, 0)),
        ],
        out_specs=pl.BlockSpec((rows_per_blk, d), lambda j, b: (b * seq_blks + j, 0)),
        out_shape=jax.ShapeDtypeStruct((batch * seq, d), gathered.dtype),
        compiler_params=pltpu.CompilerParams(
            dimension_semantics=("parallel", "parallel")
        ),
    )(gathered, W_pos, tt2, W_token_type, lnw2, lnb2)
    return out.reshape(batch, seq, d)


@jax.jit
def kernel(input_ids, token_type_ids, W_E, W_pos, W_token_type, ln_w, ln_b):
    batch, seq = input_ids.shape
    flat_idx = input_ids.reshape(batch * seq).astype(jnp.int32)
    gathered = _sc_gather(W_E, flat_idx)
    return _tc_add_ln(gathered, token_type_ids, W_pos, W_token_type, ln_w, ln_b)


# speedup vs baseline: 1.7331x; 1.0810x over previous
import functools

import jax
import jax.numpy as jnp
from jax import lax
from jax.experimental import pallas as pl
from jax.experimental.pallas import tpu as pltpu
from jax.experimental.pallas import tpu_sc as plsc

_EPS = 1e-5

_NC = 2
_NS = 16
_NW = _NC * _NS
_CHUNK = 64


def _sc_gather(table, flat_idx):
    n = flat_idx.shape[0]
    d = table.shape[1]
    b_per_w = n // _NW
    n_chunks = b_per_w // _CHUNK
    mesh = plsc.VectorSubcoreMesh(core_axis_name="c", subcore_axis_name="s")

    @functools.partial(
        pl.kernel,
        out_type=jax.ShapeDtypeStruct((n, d), table.dtype),
        mesh=mesh,
        scratch_types=[
            pltpu.VMEM((b_per_w,), jnp.int32),
            pltpu.VMEM((_CHUNK, d), jnp.float32),
            pltpu.VMEM((_CHUNK, d), jnp.float32),
            pltpu.SemaphoreType.DMA,
            pltpu.SemaphoreType.DMA,
            pltpu.SemaphoreType.DMA,
            pltpu.SemaphoreType.DMA,
        ],
    )
    def gather_kernel(table_hbm, idx_hbm, out_hbm, idx_v, rows0, rows1,
                      gsem0, gsem1, osem0, osem1):
        wid = lax.axis_index("s") * _NC + lax.axis_index("c")
        base = wid * b_per_w
        pltpu.sync_copy(idx_hbm.at[pl.ds(base, b_per_w)], idx_v)
        bufs = (rows0, rows1)
        gsems = (gsem0, gsem1)
        osems = (osem0, osem1)
        gather_h = [None, None]
        store_h = [None, None]
        gather_h[0] = pltpu.async_copy(
            table_hbm.at[idx_v.at[pl.ds(0, _CHUNK)]], bufs[0], gsems[0]
        )
        for c in range(n_chunks):
            cur = c & 1
            gather_h[cur].wait()
            if c + 1 < n_chunks:
                nb = (c + 1) & 1
                if store_h[nb] is not None:
                    store_h[nb].wait()
                gather_h[nb] = pltpu.async_copy(
                    table_hbm.at[idx_v.at[pl.ds((c + 1) * _CHUNK, _CHUNK)]],
                    bufs[nb],
                    gsems[nb],
                )
            store_h[cur] = pltpu.async_copy(
                bufs[cur], out_hbm.at[pl.ds(base + c * _CHUNK, _CHUNK)], osems[cur]
            )
        for h in store_h:
            if h is not None:
                h.wait()

    return gather_kernel(table, flat_idx)


def _ln_body(gath_ref, pos_ref, tt_ref, wtt_ref, lnw_ref, lnb_ref, out_ref):
    x = gath_ref[...]
    tt = tt_ref[...].astype(jnp.float32)
    w0 = wtt_ref[0, :][None, :]
    w1 = wtt_ref[1, :][None, :]
    tte = w0 + tt * (w1 - w0)
    x = x + pos_ref[...] + tte
    mu = jnp.mean(x, axis=-1, keepdims=True)
    xc = x - mu
    var = jnp.mean(xc * xc, axis=-1, keepdims=True)
    y = xc * lax.rsqrt(var + _EPS)
    out_ref[...] = y * lnw_ref[...] + lnb_ref[...]


def _tc_add_ln(gathered, token_type_ids, W_pos, W_token_type, ln_w, ln_b):
    batch, seq = token_type_ids.shape
    d = gathered.shape[-1]
    rows_per_blk = 1024
    seq_blks = seq // rows_per_blk

    tt2 = token_type_ids.reshape(batch * seq, 1)
    lnw2 = ln_w.reshape(1, d)
    lnb2 = ln_b.reshape(1, d)

    out = pl.pallas_call(
        _ln_body,
        grid=(seq_blks, batch),
        in_specs=[
            pl.BlockSpec((rows_per_blk, d), lambda j, b: (b * seq_blks + j, 0)),
            pl.BlockSpec((rows_per_blk, d), lambda j, b: (j, 0)),
            pl.BlockSpec((rows_per_blk, 1), lambda j, b: (b * seq_blks + j, 0)),
            pl.BlockSpec((2, d), lambda j, b: (0, 0)),
            pl.BlockSpec((1, d), lambda j, b: (0, 0)),
            pl.BlockSpec((1, d), lambda j, b: (0, 0)),
        ],
        out_specs=pl.BlockSpec((rows_per_blk, d), lambda j, b: (b * seq_blks + j, 0)),
        out_shape=jax.ShapeDtypeStruct((batch * seq, d), gathered.dtype),
        compiler_params=pltpu.CompilerParams(
            dimension_semantics=("parallel", "parallel")
        ),
    )(gathered, W_pos, tt2, W_token_type, lnw2, lnb2)
    return out.reshape(batch, seq, d)


@jax.jit
def kernel(input_ids, token_type_ids, W_E, W_pos, W_token_type, ln_w, ln_b):
    batch, seq = input_ids.shape
    flat_idx = input_ids.reshape(batch * seq).astype(jnp.int32)
    gathered = _sc_gather(W_E, flat_idx)
    return _tc_add_ln(gathered, token_type_ids, W_pos, W_token_type, ln_w, ln_b)


# speedup vs baseline: 1.7362x; 1.0018x over previous
import functools

import jax
import jax.numpy as jnp
from jax import lax
from jax.experimental import pallas as pl
from jax.experimental.pallas import tpu as pltpu
from jax.experimental.pallas import tpu_sc as plsc

_EPS = 1e-5

_NC = 2
_NS = 16
_NW = _NC * _NS
_CHUNK = 64


def _sc_gather(table, ids2d):
    n = ids2d.shape[0] * ids2d.shape[1]
    d = table.shape[1]
    b_per_w = n // _NW
    n_chunks = b_per_w // _CHUNK
    mesh = plsc.VectorSubcoreMesh(core_axis_name="c", subcore_axis_name="s")

    @functools.partial(
        pl.kernel,
        out_type=jax.ShapeDtypeStruct((n, d), table.dtype),
        mesh=mesh,
        scratch_types=[
            pltpu.VMEM((b_per_w,), jnp.int32),
            pltpu.VMEM((_CHUNK, d), jnp.float32),
            pltpu.VMEM((_CHUNK, d), jnp.float32),
            pltpu.SemaphoreType.DMA,
            pltpu.SemaphoreType.DMA,
            pltpu.SemaphoreType.DMA,
            pltpu.SemaphoreType.DMA,
        ],
    )
    def gather_kernel(table_hbm, idx_hbm, out_hbm, idx_v, rows0, rows1,
                      gsem0, gsem1, osem0, osem1):
        wid = lax.axis_index("s") * _NC + lax.axis_index("c")
        base = wid * b_per_w
        w_per_row = idx_hbm.shape[1] // b_per_w
        row = wid // w_per_row
        col = (wid % w_per_row) * b_per_w
        pltpu.sync_copy(idx_hbm.at[row, pl.ds(col, b_per_w)], idx_v)
        bufs = (rows0, rows1)
        gsems = (gsem0, gsem1)
        osems = (osem0, osem1)
        gather_h = [None, None]
        store_h = [None, None]
        gather_h[0] = pltpu.async_copy(
            table_hbm.at[idx_v.at[pl.ds(0, _CHUNK)]], bufs[0], gsems[0]
        )
        for c in range(n_chunks):
            cur = c & 1
            gather_h[cur].wait()
            if c + 1 < n_chunks:
                nb = (c + 1) & 1
                if store_h[nb] is not None:
                    store_h[nb].wait()
                gather_h[nb] = pltpu.async_copy(
                    table_hbm.at[idx_v.at[pl.ds((c + 1) * _CHUNK, _CHUNK)]],
                    bufs[nb],
                    gsems[nb],
                )
            store_h[cur] = pltpu.async_copy(
                bufs[cur], out_hbm.at[pl.ds(base + c * _CHUNK, _CHUNK)], osems[cur]
            )
        for h in store_h:
            if h is not None:
                h.wait()

    return gather_kernel(table, ids2d)


def _ln_body(gath_ref, pos_ref, tt_ref, wtt_ref, lnw_ref, lnb_ref, out_ref):
    x = gath_ref[...]
    tt = tt_ref[0].astype(jnp.float32)
    w0 = wtt_ref[0, :][None, :]
    w1 = wtt_ref[1, :][None, :]
    tte = w0 + tt * (w1 - w0)
    x = x + pos_ref[...] + tte
    mu = jnp.mean(x, axis=-1, keepdims=True)
    xc = x - mu
    var = jnp.mean(xc * xc, axis=-1, keepdims=True)
    y = xc * lax.rsqrt(var + _EPS)
    out_ref[0] = y * lnw_ref[...] + lnb_ref[...]


def _tc_add_ln(gathered, token_type_ids, W_pos, W_token_type, ln_w, ln_b):
    batch, seq = token_type_ids.shape
    d = gathered.shape[-1]
    rows_per_blk = 1024
    seq_blks = seq // rows_per_blk

    tt3 = token_type_ids.reshape(batch, seq, 1)
    lnw2 = ln_w.reshape(1, d)
    lnb2 = ln_b.reshape(1, d)

    return pl.pallas_call(
        _ln_body,
        grid=(seq_blks, batch),
        in_specs=[
            pl.BlockSpec((rows_per_blk, d), lambda j, b: (b * seq_blks + j, 0)),
            pl.BlockSpec((rows_per_blk, d), lambda j, b: (j, 0)),
            pl.BlockSpec((1, rows_per_blk, 1), lambda j, b: (b, j, 0)),
            pl.BlockSpec((2, d), lambda j, b: (0, 0)),
            pl.BlockSpec((1, d), lambda j, b: (0, 0)),
            pl.BlockSpec((1, d), lambda j, b: (0, 0)),
        ],
        out_specs=pl.BlockSpec((1, rows_per_blk, d), lambda j, b: (b, j, 0)),
        out_shape=jax.ShapeDtypeStruct((batch, seq, d), gathered.dtype),
        compiler_params=pltpu.CompilerParams(
            dimension_semantics=("parallel", "parallel")
        ),
    )(gathered, W_pos, tt3, W_token_type, lnw2, lnb2)


@jax.jit
def kernel(input_ids, token_type_ids, W_E, W_pos, W_token_type, ln_w, ln_b):
    gathered = _sc_gather(W_E, input_ids.astype(jnp.int32))
    return _tc_add_ln(gathered, token_type_ids, W_pos, W_token_type, ln_w, ln_b)
